# baseline (speedup 1.0000x reference)
import functools

import jax
import jax.numpy as jnp
from jax import lax
from jax.experimental import pallas as pl
from jax.experimental.pallas import tpu as pltpu
from jax.experimental.pallas import tpu_sc as plsc

_INT_MIN = -(2**31)


def _scores_body(x_ref, w_ref, e_ref, o_ref):
    r = jax.lax.dot_general(x_ref[...], w_ref[...], (((1,), (1,)), ((), ())),
                            preferred_element_type=jnp.float32)
    o_ref[0, 0, :] = r[:, 0] + e_ref[0, 0, :]


def _make_select_body(B, T, K, TPW):
    R = (B * T) // 128
    SPR = T // 128

    def _select_body(s_ref, o_ref):
        s = s_ref[...]
        key = lax.bitcast_convert_type(s, jnp.int32)
        key = key ^ (jnp.int32(0x7FFFFFFF) & (key >> 31))

        ri = lax.broadcasted_iota(jnp.int32, (R, 1), 0)
        grp = ri // SPR
        gid_row = lax.broadcasted_iota(jnp.int32, (B, R), 0)
        sub_row = lax.broadcasted_iota(jnp.int32, (B, R), 1) // SPR
        Gf = (gid_row == sub_row).astype(jnp.float32)
        GTf = (lax.broadcasted_iota(jnp.int32, (R, B), 0) // SPR
               == lax.broadcasted_iota(jnp.int32, (R, B), 1)).astype(jnp.float32)

        hp = jax.lax.Precision.HIGHEST

        def seg_bcast(colsum):
            g = jax.lax.dot_general(Gf, colsum, (((1,), (0,)), ((), ())),
                                    precision=hp,
                                    preferred_element_type=jnp.float32)
            return jax.lax.dot_general(GTf, g, (((1,), (0,)), ((), ())),
                                       precision=hp,
                                       preferred_element_type=jnp.float32)

        def cnt_ge(c):
            cmp = (key >= c).astype(jnp.float32)
            return seg_bcast(jnp.sum(cmp, axis=1, keepdims=True))

        zero = jnp.zeros((R, 1), jnp.int32)
        p = jnp.where(cnt_ge(zero) >= K, zero,
                      jnp.full((R, 1), _INT_MIN, jnp.int32))

        def bit_body(it, p):
            c = p + (jnp.int32(1) << (30 - it))
            return jnp.where(cnt_ge(c) >= K, c, p)

        p = lax.fori_loop(0, 31, bit_body, p)

        gt = key > p
        eq = key == p
        ngt = seg_bcast(jnp.sum(gt.astype(jnp.float32), axis=1, keepdims=True))
        need = K - ngt

        li = lax.broadcasted_iota(jnp.int32, (128, 128), 0)
        lj = lax.broadcasted_iota(jnp.int32, (128, 128), 1)
        Lincl = (li <= lj).astype(jnp.float32)
        si = lax.broadcasted_iota(jnp.int32, (R, R), 0)
        sj = lax.broadcasted_iota(jnp.int32, (R, R), 1)
        TriG = ((sj < si) & (sj // SPR == si // SPR)).astype(jnp.float32)

        def seg_prefix(m):
            pin = jax.lax.dot_general(m, Lincl, (((1,), (0,)), ((), ())),
                                      precision=hp,
                                      preferred_element_type=jnp.float32)
            rs = jnp.sum(m, axis=1, keepdims=True)
            eb = jax.lax.dot_general(TriG, rs, (((1,), (0,)), ((), ())),
                                     precision=hp,
                                     preferred_element_type=jnp.float32)
            return pin + eb

        rank = seg_prefix(eq.astype(jnp.float32))
        mask = gt | (eq & (rank <= need))
        pos = seg_prefix(mask.astype(jnp.float32))

        tflat = ri * 128 + lax.broadcasted_iota(jnp.int32, (R, 128), 1)
        gpos = jnp.where(mask,
                         grp * K + pos.astype(jnp.int32) - 1,
                         B * K + tflat // TPW)
        o_ref[...] = gpos
    return _select_body


def _make_sc_body(B, T, D, K, NC, NW):
    TPW = (B * T) // NW
    CH = 64
    NCH = TPW // CH

    def _sc_body(gpos_hbm, x_hbm, idxout_hbm, sel_hbm,
                 gpos_v, tval_v, rows_v, sem):
        wid = lax.axis_index("s") * NC + lax.axis_index("c")
        l16 = lax.iota(jnp.int32, 16)

        t0 = wid * TPW
        tloc0 = t0 - (wid // (NW // B)) * T
        for cc in range(NCH):
            pltpu.sync_copy(gpos_hbm.at[pl.ds(t0 + cc * CH, CH)],
                            gpos_v.at[cc])
        for cc in range(NCH):
            for i in range(CH // 16):
                tval_v[cc, pl.ds(i * 16, 16)] = (
                    tloc0 + cc * CH + i * 16 + l16)
        for cc in range(NCH):
            pltpu.sync_copy(tval_v.at[cc], idxout_hbm.at[gpos_v.at[cc]])
        for cc in range(NCH):
            pltpu.sync_copy(x_hbm.at[pl.ds(t0 + cc * CH, CH)], rows_v)
            pltpu.sync_copy(rows_v, sel_hbm.at[gpos_v.at[cc]])
    return _sc_body


def kernel(x, entropy_signal, W):
    B, T, D = x.shape
    K = max(1, T // 2)
    x2 = x.reshape(B * T, D)

    RB = 2048
    nblk = (B * T) // RB
    e3 = entropy_signal.reshape(nblk, 1, RB)
    W8 = jnp.broadcast_to(W, (8, D))
    scores3 = pl.pallas_call(
        _scores_body,
        grid=(nblk,),
        in_specs=[
            pl.BlockSpec((RB, D), lambda j: (j, 0)),
            pl.BlockSpec((8, D), lambda j: (0, 0)),
            pl.BlockSpec((1, 1, RB), lambda j: (j, 0, 0)),
        ],
        out_specs=pl.BlockSpec((1, 1, RB), lambda j: (j, 0, 0)),
        out_shape=jax.ShapeDtypeStruct((nblk, 1, RB), jnp.float32),
    )(x2, W8, e3)
    scores = scores3.reshape(B, T)

    info = plsc.get_sparse_core_info()
    NC, NS = info.num_cores, info.num_subcores
    NW = NC * NS
    TPW = (B * T) // NW
    R = (B * T) // 128
    gpos = pl.pallas_call(
        _make_select_body(B, T, K, TPW),
        out_shape=jax.ShapeDtypeStruct((R, 128), jnp.int32),
    )(scores.reshape(R, 128))

    mesh = plsc.VectorSubcoreMesh(core_axis_name="c", subcore_axis_name="s")
    CH = 64
    sc_fn = pl.kernel(
        _make_sc_body(B, T, D, K, NC, NW),
        out_type=[jax.ShapeDtypeStruct((B * K + NW,), jnp.int32),
                  jax.ShapeDtypeStruct((B * K + NW, D), jnp.float32)],
        mesh=mesh,
        scratch_types=[
            pltpu.VMEM((TPW // CH, CH), jnp.int32),
            pltpu.VMEM((TPW // CH, CH), jnp.int32),
            pltpu.VMEM((CH, D), jnp.float32),
            pltpu.SemaphoreType.DMA,
        ],
    )
    idxout, sel = sc_fn(gpos.reshape(B * T), x2)
    return (sel[: B * K].reshape(B, K, D),
            idxout[: B * K].reshape(B, K), scores)


# speedup vs baseline: 9.9173x; 9.9173x over previous
import functools

import jax
import jax.numpy as jnp
from jax import lax
from jax.experimental import pallas as pl
from jax.experimental.pallas import tpu as pltpu
from jax.experimental.pallas import tpu_sc as plsc

_INT_MIN = -(2**31)


def _scores_body(x_ref, w_ref, e_ref, o_ref):
    r = jax.lax.dot_general(x_ref[...], w_ref[...], (((1,), (1,)), ((), ())),
                            preferred_element_type=jnp.float32)
    o_ref[0, 0, :] = r[:, 0] + e_ref[0, 0, :]


def _make_select_body(B, T, K, TPW):
    R = (B * T) // 128
    SPR = T // 128

    def _select_body(s_ref, o_ref):
        s = s_ref[...]
        key = lax.bitcast_convert_type(s, jnp.int32)
        key = key ^ (jnp.int32(0x7FFFFFFF) & (key >> 31))

        ri = lax.broadcasted_iota(jnp.int32, (R, 1), 0)
        grp = ri // SPR
        gid_row = lax.broadcasted_iota(jnp.int32, (B, R), 0)
        sub_row = lax.broadcasted_iota(jnp.int32, (B, R), 1) // SPR
        Gf = (gid_row == sub_row).astype(jnp.float32)
        GTf = (lax.broadcasted_iota(jnp.int32, (R, B), 0) // SPR
               == lax.broadcasted_iota(jnp.int32, (R, B), 1)).astype(jnp.float32)

        hp = jax.lax.Precision.HIGHEST

        def seg_bcast(colsum):
            g = jax.lax.dot_general(Gf, colsum, (((1,), (0,)), ((), ())),
                                    precision=hp,
                                    preferred_element_type=jnp.float32)
            return jax.lax.dot_general(GTf, g, (((1,), (0,)), ((), ())),
                                       precision=hp,
                                       preferred_element_type=jnp.float32)

        def cnt_ge(c):
            cmp = (key >= c).astype(jnp.float32)
            return seg_bcast(jnp.sum(cmp, axis=1, keepdims=True))

        zero = jnp.zeros((R, 1), jnp.int32)
        p = jnp.where(cnt_ge(zero) >= K, zero,
                      jnp.full((R, 1), _INT_MIN, jnp.int32))

        def bit_body(it, p):
            c = p + (jnp.int32(1) << (30 - it))
            return jnp.where(cnt_ge(c) >= K, c, p)

        p = lax.fori_loop(0, 31, bit_body, p)

        gt = key > p
        eq = key == p
        ngt = seg_bcast(jnp.sum(gt.astype(jnp.float32), axis=1, keepdims=True))
        need = K - ngt

        li = lax.broadcasted_iota(jnp.int32, (128, 128), 0)
        lj = lax.broadcasted_iota(jnp.int32, (128, 128), 1)
        Lincl = (li <= lj).astype(jnp.float32)
        si = lax.broadcasted_iota(jnp.int32, (R, R), 0)
        sj = lax.broadcasted_iota(jnp.int32, (R, R), 1)
        TriG = ((sj < si) & (sj // SPR == si // SPR)).astype(jnp.float32)

        def seg_prefix(m):
            pin = jax.lax.dot_general(m, Lincl, (((1,), (0,)), ((), ())),
                                      precision=hp,
                                      preferred_element_type=jnp.float32)
            rs = jnp.sum(m, axis=1, keepdims=True)
            eb = jax.lax.dot_general(TriG, rs, (((1,), (0,)), ((), ())),
                                     precision=hp,
                                     preferred_element_type=jnp.float32)
            return pin + eb

        rank = seg_prefix(eq.astype(jnp.float32))
        mask = gt | (eq & (rank <= need))
        pos = seg_prefix(mask.astype(jnp.float32))

        gpos = jnp.where(mask,
                         grp * K + pos.astype(jnp.int32) - 1,
                         jnp.full((R, 128), -1, jnp.int32))
        o_ref[...] = gpos
    return _select_body


def _make_sc_scatter_body(B, T, K, NC, NW):
    TPW = (B * T) // NW
    SCH = 128
    NSC = TPW // SCH

    def _body(gpos_hbm, gidx_hbm, gpos_v, tval_v, sem):
        wid = lax.axis_index("s") * NC + lax.axis_index("c")
        l16 = lax.iota(jnp.int32, 16)
        t0 = wid * TPW
        for cc in range(NSC):
            pltpu.sync_copy(gpos_hbm.at[pl.ds(t0 + cc * SCH, SCH)],
                            gpos_v.at[cc])
        for cc in range(NSC):
            for i in range(SCH // 16):
                tval_v[cc, pl.ds(i * 16, 16)] = t0 + cc * SCH + i * 16 + l16
        for cc in range(NSC):
            pltpu.sync_copy(
                tval_v.at[cc],
                gidx_hbm.at[plsc.Indices(gpos_v.at[cc], ignored_value=-1)])
    return _body


def _make_sc_gather_body(B, T, D, K, NC, NW):
    RPW = (B * K) // NW
    GCH = 32
    NGC = RPW // GCH

    def _body(gidx_hbm, x_hbm, idxout_hbm, sel_hbm,
              gidx_v, loc_v, rows0_v, rows1_v, sem0, sem1):
        wid = lax.axis_index("s") * NC + lax.axis_index("c")
        o0 = wid * RPW
        boff = (wid // (NW // B)) * T
        for cc in range(NGC):
            pltpu.sync_copy(gidx_hbm.at[pl.ds(o0 + cc * GCH, GCH)],
                            gidx_v.at[cc])
        for cc in range(NGC):
            for i in range(GCH // 16):
                loc_v[cc, pl.ds(i * 16, 16)] = (
                    gidx_v[cc, pl.ds(i * 16, 16)] - boff)
        for cc in range(NGC):
            pltpu.sync_copy(loc_v.at[cc],
                            idxout_hbm.at[pl.ds(o0 + cc * GCH, GCH)])

        rows = (rows0_v, rows1_v)
        sems = (sem0, sem1)
        copies = [None] * NGC
        copies[0] = pltpu.async_copy(x_hbm.at[gidx_v.at[0]], rows0_v, sem0)
        for cc in range(NGC):
            if cc + 1 < NGC:
                copies[cc + 1] = pltpu.async_copy(
                    x_hbm.at[gidx_v.at[cc + 1]],
                    rows[(cc + 1) % 2], sems[(cc + 1) % 2])
            copies[cc].wait()
            pltpu.sync_copy(rows[cc % 2],
                            sel_hbm.at[pl.ds(o0 + cc * GCH, GCH)])
    return _body


def kernel(x, entropy_signal, W):
    B, T, D = x.shape
    K = max(1, T // 2)
    x2 = x.reshape(B * T, D)

    RB = 2048
    nblk = (B * T) // RB
    e3 = entropy_signal.reshape(nblk, 1, RB)
    W8 = jnp.broadcast_to(W, (8, D))
    scores3 = pl.pallas_call(
        _scores_body,
        grid=(nblk,),
        in_specs=[
            pl.BlockSpec((RB, D), lambda j: (j, 0)),
            pl.BlockSpec((8, D), lambda j: (0, 0)),
            pl.BlockSpec((1, 1, RB), lambda j: (j, 0, 0)),
        ],
        out_specs=pl.BlockSpec((1, 1, RB), lambda j: (j, 0, 0)),
        out_shape=jax.ShapeDtypeStruct((nblk, 1, RB), jnp.float32),
    )(x2, W8, e3)
    scores = scores3.reshape(B, T)

    info = plsc.get_sparse_core_info()
    NC, NS = info.num_cores, info.num_subcores
    NW = NC * NS
    TPW = (B * T) // NW
    R = (B * T) // 128
    gpos = pl.pallas_call(
        _make_select_body(B, T, K, TPW),
        out_shape=jax.ShapeDtypeStruct((R, 128), jnp.int32),
    )(scores.reshape(R, 128))

    mesh = plsc.VectorSubcoreMesh(core_axis_name="c", subcore_axis_name="s")
    RPW = (B * K) // NW
    SCH, GCH = 128, 32
    scatter_fn = pl.kernel(
        _make_sc_scatter_body(B, T, K, NC, NW),
        out_type=jax.ShapeDtypeStruct((B * K,), jnp.int32),
        mesh=mesh,
        scratch_types=[
            pltpu.VMEM((TPW // SCH, SCH), jnp.int32),
            pltpu.VMEM((TPW // SCH, SCH), jnp.int32),
            pltpu.SemaphoreType.DMA,
        ],
    )
    gidx = scatter_fn(gpos.reshape(B * T))

    gather_fn = pl.kernel(
        _make_sc_gather_body(B, T, D, K, NC, NW),
        out_type=[jax.ShapeDtypeStruct((B * K,), jnp.int32),
                  jax.ShapeDtypeStruct((B * K, D), jnp.float32)],
        mesh=mesh,
        scratch_types=[
            pltpu.VMEM((RPW // GCH, GCH), jnp.int32),
            pltpu.VMEM((RPW // GCH, GCH), jnp.int32),
            pltpu.VMEM((GCH, D), jnp.float32),
            pltpu.VMEM((GCH, D), jnp.float32),
            pltpu.SemaphoreType.DMA,
            pltpu.SemaphoreType.DMA,
        ],
    )
    idxout, sel = gather_fn(gidx, x2)
    return sel.reshape(B, K, D), idxout.reshape(B, K), scores
